# baseline (speedup 1.0000x reference)
import functools

import jax
import jax.numpy as jnp
from jax import lax
from jax.experimental import pallas as pl
from jax.experimental.pallas import tpu as pltpu
from jax.experimental.pallas import tpu_sc as plsc

_N = 10000
_E = 320000
_D = 128
_H = 128
_C = 64
_W = 64

_CHUNK = 128
_NW = 32
_CPW = 80
_NCH = _CPW * _NW
_EP = _NCH * _CHUNK
_NPAD = 16
_NA = _N + _NPAD
_RPT = _NA // 16

_SELU_ALPHA = 1.6732632423543772
_SELU_SCALE = 1.0507009873554805


def _make_sc_agg(width, nb):
  _NB = nb
  _W2 = width
  out_types = [jax.ShapeDtypeStruct((2, _NA, _W2), jnp.float32)]
  scratch = [
      pltpu.VMEM_SHARED((_NA, _W2), jnp.float32),
      pltpu.VMEM((_NB, 2, _CHUNK), jnp.int32),
      pltpu.VMEM((_NB, _CHUNK, _W2), jnp.float32),
      pltpu.VMEM((_RPT, _W2), jnp.float32),
      pltpu.SemaphoreType.DMA((_NB,)),
      pltpu.SemaphoreType.DMA((_NB,)),
      pltpu.SemaphoreType.DMA((_NB,)),
  ]

  mesh = plsc.VectorSubcoreMesh(core_axis_name="c", subcore_axis_name="s")

  @functools.partial(
      pl.kernel, mesh=mesh, out_type=tuple(out_types), scratch_types=scratch,
      compiler_params=pltpu.CompilerParams(use_tc_tiling_on_sc=False))
  def body(*refs):
    (table_hbm, eidx_hbm, acc_out, acc_sh, idxs_v, rows_v,
     zbuf, ldsem, gsem, asem) = refs

    cid = lax.axis_index("c")
    sid = lax.axis_index("s")
    w = sid * 2 + cid
    base = w * _CPW

    for b in range(_NB):
      e0 = (base + b) * _CHUNK
      pltpu.async_copy(
          eidx_hbm.at[0, pl.ds(e0, _CHUNK)], idxs_v.at[b, 0], ldsem.at[b])
      pltpu.async_copy(
          eidx_hbm.at[1, pl.ds(e0, _CHUNK)], idxs_v.at[b, 1], ldsem.at[b])

    zero16 = jnp.zeros((16,), jnp.float32)

    def zfill(r, _):
      for j in range(_W2 // 16):
        zbuf[r, pl.ds(j * 16, 16)] = zero16
      return 0

    lax.fori_loop(0, _RPT, zfill, 0)

    row0 = sid * _RPT
    pltpu.sync_copy(zbuf, acc_sh.at[pl.ds(row0, _RPT)])

    plsc.subcore_barrier()

    _K = _CPW // _NB

    def outer(k, _):
      c0 = base + k * _NB
      for b in range(_NB):
        e0 = (c0 + b) * _CHUNK
        pltpu.make_async_copy(
            eidx_hbm.at[0, pl.ds(e0, _CHUNK)], idxs_v.at[b, 0],
            ldsem.at[b]).wait()
        pltpu.make_async_copy(
            eidx_hbm.at[1, pl.ds(e0, _CHUNK)], idxs_v.at[b, 1],
            ldsem.at[b]).wait()
        pltpu.async_copy(
            table_hbm.at[idxs_v.at[b, 0]], rows_v.at[b], gsem.at[b])
      for b in range(_NB):
        pltpu.make_async_copy(
            table_hbm.at[idxs_v.at[b, 0]], rows_v.at[b], gsem.at[b]).wait()
        pltpu.async_copy(
            rows_v.at[b], acc_sh.at[idxs_v.at[b, 1]], asem.at[b], add=True)
      for b in range(_NB):
        pltpu.make_async_copy(
            rows_v.at[b], acc_sh.at[idxs_v.at[b, 1]], asem.at[b]).wait()

        @pl.when(k < _K - 1)
        def _():
          e1 = (c0 + _NB + b) * _CHUNK
          pltpu.async_copy(
              eidx_hbm.at[0, pl.ds(e1, _CHUNK)], idxs_v.at[b, 0],
              ldsem.at[b])
          pltpu.async_copy(
              eidx_hbm.at[1, pl.ds(e1, _CHUNK)], idxs_v.at[b, 1],
              ldsem.at[b])

      return 0

    lax.fori_loop(0, _K, outer, 0)
    plsc.subcore_barrier()

    pltpu.sync_copy(acc_sh.at[pl.ds(row0, _RPT)], zbuf)
    pltpu.sync_copy(zbuf, acc_out.at[cid, pl.ds(row0, _RPT)])

  return body


_BLK = 1000


def _mm2_body(x_ref, wn_ref, xna_ref, xnb_ref):
  xn = jnp.dot(x_ref[...], wn_ref[...], preferred_element_type=jnp.float32)
  xna_ref[...] = jnp.concatenate(
      [xn[:, :_W], jnp.ones((_BLK, 16), jnp.float32)], axis=1)
  xnb_ref[...] = xn[:, _W:]


def _mid_body(x_ref, ws1_ref, b1_ref, acca_ref, accb_ref,
              ws2_ref, wn2_ref, b2_ref, hs2_ref, hn2_ref):
  agga = acca_ref[0, :, :_W] + acca_ref[1, :, :_W]
  aggb = accb_ref[0] + accb_ref[1]
  deg = acca_ref[0, :, _W:_W + 1] + acca_ref[1, :, _W:_W + 1]
  dinv = 1.0 / jnp.maximum(deg, 1.0)
  agg = jnp.concatenate([agga, aggb], axis=1)
  h = (jnp.dot(x_ref[...], ws1_ref[...], preferred_element_type=jnp.float32)
       + b1_ref[...] + agg * dinv)
  h = _SELU_SCALE * jnp.where(
      h > 0, h, _SELU_ALPHA * (jnp.exp(jnp.minimum(h, 0.0)) - 1.0))
  hs2_ref[...] = (
      jnp.dot(h, ws2_ref[...], preferred_element_type=jnp.float32)
      + b2_ref[...])
  hn2_ref[...] = jnp.dot(h, wn2_ref[...], preferred_element_type=jnp.float32)


def _out_body(hs2_ref, acc2_ref, deg_ref, o_ref):
  deg = deg_ref[0, :, 0:1] + deg_ref[1, :, 0:1]
  dinv = 1.0 / jnp.maximum(deg, 1.0)
  z = hs2_ref[...] + (acc2_ref[0] + acc2_ref[1]) * dinv
  m = jnp.max(z, axis=1, keepdims=True)
  e = jnp.exp(z - m)
  o_ref[...] = e / jnp.sum(e, axis=1, keepdims=True)


def _row_spec(w):
  return pl.BlockSpec((_BLK, w), lambda i: (i, 0))


def _part_spec(w):
  return pl.BlockSpec((2, _BLK, w), lambda i: (0, i, 0))


def _full_spec(r, c):
  return pl.BlockSpec((r, c), lambda i: (0, 0))


def kernel(x, edge_index, W_self1, W_neigh1, b1, W_self2, W_neigh2, b2):
  npad = _EP - _E
  pad_src = (jnp.arange(npad, dtype=jnp.int32) * 1009) % _N
  pad_dst = _N + (jnp.arange(npad, dtype=jnp.int32) % _NPAD)
  eidx = jnp.concatenate(
      [edge_index, jnp.stack([pad_src, pad_dst])], axis=1)

  xn1a, xn1b = pl.pallas_call(
      _mm2_body,
      grid=(_N // _BLK,),
      in_specs=[_row_spec(_D), _full_spec(_D, _H)],
      out_specs=[_row_spec(_W + 16), _row_spec(_W)],
      out_shape=[jax.ShapeDtypeStruct((_N, _W + 16), jnp.float32),
                 jax.ShapeDtypeStruct((_N, _W), jnp.float32)],
  )(x, W_neigh1)

  agg_deg = _make_sc_agg(_W + 16, 2)
  agg_plain = _make_sc_agg(_W, 4)
  (accpa,) = agg_deg(xn1a, eidx)
  (accpb,) = agg_plain(xn1b, eidx)
  degp = accpa[:, :, _W:_W + 8]

  hs2, hn2 = pl.pallas_call(
      _mid_body,
      grid=(_N // _BLK,),
      in_specs=[_row_spec(_D), _full_spec(_D, _H), _full_spec(1, _H),
                _part_spec(_W + 16), _part_spec(_W),
                _full_spec(_H, _C), _full_spec(_H, _C), _full_spec(1, _C)],
      out_specs=[_row_spec(_C), _row_spec(_C)],
      out_shape=[jax.ShapeDtypeStruct((_N, _C), jnp.float32),
                 jax.ShapeDtypeStruct((_N, _C), jnp.float32)],
  )(x, W_self1, b1.reshape(1, _H), accpa, accpb,
    W_self2, W_neigh2, b2.reshape(1, _C))

  (accp2,) = agg_plain(hn2, eidx)

  out = pl.pallas_call(
      _out_body,
      grid=(_N // _BLK,),
      in_specs=[_row_spec(_C), _part_spec(_C), _part_spec(8)],
      out_specs=_row_spec(_C),
      out_shape=jax.ShapeDtypeStruct((_N, _C), jnp.float32),
  )(hs2, accp2, degp)
  return out


# speedup vs baseline: 1.0462x; 1.0462x over previous
import functools

import jax
import jax.numpy as jnp
from jax import lax
from jax.experimental import pallas as pl
from jax.experimental.pallas import tpu as pltpu
from jax.experimental.pallas import tpu_sc as plsc

_N = 10000
_E = 320000
_D = 128
_H = 128
_C = 64
_W = 64

_CHUNK = 128
_NW = 32
_CPW = 80
_NCH = _CPW * _NW
_EP = _NCH * _CHUNK
_NPAD = 16
_NA = _N + _NPAD
_RPT = _NA // 16

_SELU_ALPHA = 1.6732632423543772
_SELU_SCALE = 1.0507009873554805


def _make_sc_agg(with_deg, nb):
  _NB = nb
  out_types = [jax.ShapeDtypeStruct((2, _NA, _W), jnp.float32)]
  scratch = [
      pltpu.VMEM_SHARED((_NA, _W), jnp.float32),
      pltpu.VMEM((_NB, 2, _CHUNK), jnp.int32),
      pltpu.VMEM((_NB, _CHUNK, _W), jnp.float32),
      pltpu.VMEM((_RPT, _W), jnp.float32),
      pltpu.SemaphoreType.DMA((_NB,)),
      pltpu.SemaphoreType.DMA((_NB,)),
      pltpu.SemaphoreType.DMA((_NB,)),
  ]
  if with_deg:
    out_types.append(jax.ShapeDtypeStruct((2, _NA, 8), jnp.float32))
    scratch += [
        pltpu.VMEM_SHARED((_NA, 8), jnp.float32),
        pltpu.VMEM((_CHUNK, 8), jnp.float32),
        pltpu.VMEM((_RPT, 8), jnp.float32),
        pltpu.SemaphoreType.DMA((_NB,)),
    ]

  mesh = plsc.VectorSubcoreMesh(core_axis_name="c", subcore_axis_name="s")

  @functools.partial(
      pl.kernel, mesh=mesh, out_type=tuple(out_types), scratch_types=scratch,
      compiler_params=pltpu.CompilerParams(use_tc_tiling_on_sc=False))
  def body(*refs):
    if with_deg:
      (table_hbm, eidx_hbm, aux_hbm, acc_out, deg_out, acc_sh,
       idxs_v, rows_v, zbuf, ldsem, gsem, asem, deg_sh, ones_v, zbuf8,
       dsem) = refs
    else:
      (table_hbm, eidx_hbm, acc_out, acc_sh, idxs_v, rows_v,
       zbuf, ldsem, gsem, asem) = refs

    cid = lax.axis_index("c")
    sid = lax.axis_index("s")
    w = sid * 2 + cid
    base = w * _CPW

    for b in range(_NB):
      e0 = (base + b) * _CHUNK
      pltpu.async_copy(
          eidx_hbm.at[0, pl.ds(e0, _CHUNK)], idxs_v.at[b, 0], ldsem.at[b])
      pltpu.async_copy(
          eidx_hbm.at[1, pl.ds(e0, _CHUNK)], idxs_v.at[b, 1], ldsem.at[b])

    zero16 = jnp.zeros((16,), jnp.float32)

    def zfill(r, _):
      for j in range(_W // 16):
        zbuf[r, pl.ds(j * 16, 16)] = zero16
      return 0

    lax.fori_loop(0, _RPT, zfill, 0)

    row0 = sid * _RPT
    pltpu.sync_copy(zbuf, acc_sh.at[pl.ds(row0, _RPT)])
    if with_deg:
      pltpu.sync_copy(aux_hbm.at[pl.ds(_RPT, _CHUNK)], ones_v)
      pltpu.sync_copy(aux_hbm.at[pl.ds(0, _RPT)], zbuf8)
      pltpu.sync_copy(zbuf8, deg_sh.at[pl.ds(row0, _RPT)])

    plsc.subcore_barrier()

    _K = _CPW // _NB

    def outer(k, _):
      c0 = base + k * _NB
      for b in range(_NB):
        e0 = (c0 + b) * _CHUNK
        pltpu.make_async_copy(
            eidx_hbm.at[0, pl.ds(e0, _CHUNK)], idxs_v.at[b, 0],
            ldsem.at[b]).wait()
        pltpu.make_async_copy(
            eidx_hbm.at[1, pl.ds(e0, _CHUNK)], idxs_v.at[b, 1],
            ldsem.at[b]).wait()
        pltpu.async_copy(
            table_hbm.at[idxs_v.at[b, 0]], rows_v.at[b], gsem.at[b])
      for b in range(_NB):
        pltpu.make_async_copy(
            table_hbm.at[idxs_v.at[b, 0]], rows_v.at[b], gsem.at[b]).wait()
        pltpu.async_copy(
            rows_v.at[b], acc_sh.at[idxs_v.at[b, 1]], asem.at[b], add=True)
        if with_deg:
          pltpu.async_copy(
              ones_v, deg_sh.at[idxs_v.at[b, 1]], dsem.at[b], add=True)
      for b in range(_NB):
        pltpu.make_async_copy(
            rows_v.at[b], acc_sh.at[idxs_v.at[b, 1]], asem.at[b]).wait()
        if with_deg:
          pltpu.make_async_copy(
              ones_v, deg_sh.at[idxs_v.at[b, 1]], dsem.at[b]).wait()

        @pl.when(k < _K - 1)
        def _():
          e1 = (c0 + _NB + b) * _CHUNK
          pltpu.async_copy(
              eidx_hbm.at[0, pl.ds(e1, _CHUNK)], idxs_v.at[b, 0],
              ldsem.at[b])
          pltpu.async_copy(
              eidx_hbm.at[1, pl.ds(e1, _CHUNK)], idxs_v.at[b, 1],
              ldsem.at[b])

      return 0

    lax.fori_loop(0, _K, outer, 0)
    plsc.subcore_barrier()

    pltpu.sync_copy(acc_sh.at[pl.ds(row0, _RPT)], zbuf)
    pltpu.sync_copy(zbuf, acc_out.at[cid, pl.ds(row0, _RPT)])
    if with_deg:
      pltpu.sync_copy(deg_sh.at[pl.ds(row0, _RPT)], zbuf8)
      pltpu.sync_copy(zbuf8, deg_out.at[cid, pl.ds(row0, _RPT)])

  return body


_BLK = 2000


def _mm2_body(x_ref, wn_ref, xna_ref, xnb_ref):
  xn = jnp.dot(x_ref[...], wn_ref[...], preferred_element_type=jnp.float32)
  xna_ref[...] = xn[:, :_W]
  xnb_ref[...] = xn[:, _W:]


def _mid_body(x_ref, ws1_ref, b1_ref, acca_ref, accb_ref, deg_ref,
              ws2_ref, wn2_ref, b2_ref, hs2_ref, hn2_ref):
  agga = acca_ref[0] + acca_ref[1]
  aggb = accb_ref[0] + accb_ref[1]
  deg = deg_ref[0, :, 0:1] + deg_ref[1, :, 0:1]
  dinv = 1.0 / jnp.maximum(deg, 1.0)
  agg = jnp.concatenate([agga, aggb], axis=1)
  h = (jnp.dot(x_ref[...], ws1_ref[...], preferred_element_type=jnp.float32)
       + b1_ref[...] + agg * dinv)
  h = _SELU_SCALE * jnp.where(
      h > 0, h, _SELU_ALPHA * (jnp.exp(jnp.minimum(h, 0.0)) - 1.0))
  hs2_ref[...] = (
      jnp.dot(h, ws2_ref[...], preferred_element_type=jnp.float32)
      + b2_ref[...])
  hn2_ref[...] = jnp.dot(h, wn2_ref[...], preferred_element_type=jnp.float32)


def _out_body(hs2_ref, acc2_ref, deg_ref, o_ref):
  deg = deg_ref[0, :, 0:1] + deg_ref[1, :, 0:1]
  dinv = 1.0 / jnp.maximum(deg, 1.0)
  z = hs2_ref[...] + (acc2_ref[0] + acc2_ref[1]) * dinv
  m = jnp.max(z, axis=1, keepdims=True)
  e = jnp.exp(z - m)
  o_ref[...] = e / jnp.sum(e, axis=1, keepdims=True)


def _row_spec(w):
  return pl.BlockSpec((_BLK, w), lambda i: (i, 0))


def _part_spec(w):
  return pl.BlockSpec((2, _BLK, w), lambda i: (0, i, 0))


def _full_spec(r, c):
  return pl.BlockSpec((r, c), lambda i: (0, 0))


def kernel(x, edge_index, W_self1, W_neigh1, b1, W_self2, W_neigh2, b2):
  npad = _EP - _E
  pad_src = (jnp.arange(npad, dtype=jnp.int32) * 1009) % _N
  pad_dst = _N + (jnp.arange(npad, dtype=jnp.int32) % _NPAD)
  eidx = jnp.concatenate(
      [edge_index, jnp.stack([pad_src, pad_dst])], axis=1)

  xn1a, xn1b = pl.pallas_call(
      _mm2_body,
      grid=(_N // _BLK,),
      in_specs=[_row_spec(_D), _full_spec(_D, _H)],
      out_specs=[_row_spec(_W), _row_spec(_W)],
      out_shape=[jax.ShapeDtypeStruct((_N, _W), jnp.float32),
                 jax.ShapeDtypeStruct((_N, _W), jnp.float32)],
  )(x, W_neigh1)

  aux = jnp.concatenate([jnp.zeros((_RPT, 8), jnp.float32),
                         jnp.ones((_CHUNK, 8), jnp.float32)])
  agg_deg = _make_sc_agg(True, 2)
  agg_plain = _make_sc_agg(False, 4)
  accpa, degp = agg_deg(xn1a, eidx, aux)
  (accpb,) = agg_plain(xn1b, eidx)

  hs2, hn2 = pl.pallas_call(
      _mid_body,
      grid=(_N // _BLK,),
      in_specs=[_row_spec(_D), _full_spec(_D, _H), _full_spec(1, _H),
                _part_spec(_W), _part_spec(_W), _part_spec(8),
                _full_spec(_H, _C), _full_spec(_H, _C), _full_spec(1, _C)],
      out_specs=[_row_spec(_C), _row_spec(_C)],
      out_shape=[jax.ShapeDtypeStruct((_N, _C), jnp.float32),
                 jax.ShapeDtypeStruct((_N, _C), jnp.float32)],
  )(x, W_self1, b1.reshape(1, _H), accpa, accpb, degp,
    W_self2, W_neigh2, b2.reshape(1, _C))

  (accp2,) = agg_plain(hn2, eidx)

  out = pl.pallas_call(
      _out_body,
      grid=(_N // _BLK,),
      in_specs=[_row_spec(_C), _part_spec(_C), _part_spec(8)],
      out_specs=_row_spec(_C),
      out_shape=jax.ShapeDtypeStruct((_N, _C), jnp.float32),
  )(hs2, accp2, degp)
  return out


# speedup vs baseline: 1.0517x; 1.0052x over previous
import functools

import jax
import jax.numpy as jnp
from jax import lax
from jax.experimental import pallas as pl
from jax.experimental.pallas import tpu as pltpu
from jax.experimental.pallas import tpu_sc as plsc

_N = 10000
_E = 320000
_D = 128
_H = 128
_C = 64
_W = 64

_CHUNK = 128
_NW = 32
_CPW = 80
_NCH = _CPW * _NW
_EP = _NCH * _CHUNK
_NPAD = 16
_NA = _N + _NPAD
_RPT = _NA // 16

_SELU_ALPHA = 1.6732632423543772
_SELU_SCALE = 1.0507009873554805


def _make_sc_agg(with_deg, nb):
  _NB = nb
  out_types = [jax.ShapeDtypeStruct((2, _NA, _W), jnp.float32)]
  scratch = [
      pltpu.VMEM_SHARED((_NA, _W), jnp.float32),
      pltpu.VMEM((_NB, 2, _CHUNK), jnp.int32),
      pltpu.VMEM((_NB, _CHUNK, _W), jnp.float32),
      pltpu.VMEM((_RPT, _W), jnp.float32),
      pltpu.SemaphoreType.DMA((_NB,)),
      pltpu.SemaphoreType.DMA((_NB,)),
      pltpu.SemaphoreType.DMA((_NB,)),
  ]
  if with_deg:
    out_types.append(jax.ShapeDtypeStruct((2, _NA, 8), jnp.float32))
    scratch += [
        pltpu.VMEM_SHARED((_NA, 8), jnp.float32),
        pltpu.VMEM((_CHUNK, 8), jnp.float32),
        pltpu.VMEM((_RPT, 8), jnp.float32),
        pltpu.SemaphoreType.DMA((_NB,)),
    ]

  mesh = plsc.VectorSubcoreMesh(core_axis_name="c", subcore_axis_name="s")

  @functools.partial(
      pl.kernel, mesh=mesh, out_type=tuple(out_types), scratch_types=scratch,
      compiler_params=pltpu.CompilerParams(use_tc_tiling_on_sc=False))
  def body(*refs):
    if with_deg:
      (table_hbm, eidx_hbm, aux_hbm, acc_out, deg_out, acc_sh,
       idxs_v, rows_v, zbuf, ldsem, gsem, asem, deg_sh, ones_v, zbuf8,
       dsem) = refs
    else:
      (table_hbm, eidx_hbm, acc_out, acc_sh, idxs_v, rows_v,
       zbuf, ldsem, gsem, asem) = refs

    cid = lax.axis_index("c")
    sid = lax.axis_index("s")
    w = sid * 2 + cid
    base = w * _CPW

    for b in range(_NB):
      e0 = (base + b) * _CHUNK
      pltpu.async_copy(
          eidx_hbm.at[0, pl.ds(e0, _CHUNK)], idxs_v.at[b, 0], ldsem.at[b])
      pltpu.async_copy(
          eidx_hbm.at[1, pl.ds(e0, _CHUNK)], idxs_v.at[b, 1], ldsem.at[b])

    zero16 = jnp.zeros((16,), jnp.float32)

    def zfill(r, _):
      for j in range(_W // 16):
        zbuf[r, pl.ds(j * 16, 16)] = zero16
      return 0

    lax.fori_loop(0, _RPT, zfill, 0)

    row0 = sid * _RPT
    pltpu.sync_copy(zbuf, acc_sh.at[pl.ds(row0, _RPT)])
    if with_deg:
      pltpu.sync_copy(aux_hbm.at[pl.ds(_RPT, _CHUNK)], ones_v)
      pltpu.sync_copy(aux_hbm.at[pl.ds(0, _RPT)], zbuf8)
      pltpu.sync_copy(zbuf8, deg_sh.at[pl.ds(row0, _RPT)])

    plsc.subcore_barrier()

    _K = _CPW // _NB

    def outer(k, _):
      c0 = base + k * _NB
      for b in range(_NB):
        e0 = (c0 + b) * _CHUNK
        pltpu.make_async_copy(
            eidx_hbm.at[0, pl.ds(e0, _CHUNK)], idxs_v.at[b, 0],
            ldsem.at[b]).wait()
        pltpu.make_async_copy(
            eidx_hbm.at[1, pl.ds(e0, _CHUNK)], idxs_v.at[b, 1],
            ldsem.at[b]).wait()
        pltpu.async_copy(
            table_hbm.at[idxs_v.at[b, 0]], rows_v.at[b], gsem.at[b])
      for b in range(_NB):
        pltpu.make_async_copy(
            table_hbm.at[idxs_v.at[b, 0]], rows_v.at[b], gsem.at[b]).wait()
        pltpu.async_copy(
            rows_v.at[b], acc_sh.at[idxs_v.at[b, 1]], asem.at[b], add=True)
        if with_deg:
          pltpu.async_copy(
              ones_v, deg_sh.at[idxs_v.at[b, 1]], dsem.at[b], add=True)
      for b in range(_NB):
        pltpu.make_async_copy(
            rows_v.at[b], acc_sh.at[idxs_v.at[b, 1]], asem.at[b]).wait()
        if with_deg:
          pltpu.make_async_copy(
              ones_v, deg_sh.at[idxs_v.at[b, 1]], dsem.at[b]).wait()

        @pl.when(k < _K - 1)
        def _():
          e1 = (c0 + _NB + b) * _CHUNK
          pltpu.async_copy(
              eidx_hbm.at[0, pl.ds(e1, _CHUNK)], idxs_v.at[b, 0],
              ldsem.at[b])
          pltpu.async_copy(
              eidx_hbm.at[1, pl.ds(e1, _CHUNK)], idxs_v.at[b, 1],
              ldsem.at[b])

      return 0

    lax.fori_loop(0, _K, outer, 0)
    plsc.subcore_barrier()

    pltpu.sync_copy(acc_sh.at[pl.ds(row0, _RPT)], zbuf)
    pltpu.sync_copy(zbuf, acc_out.at[cid, pl.ds(row0, _RPT)])
    if with_deg:
      pltpu.sync_copy(deg_sh.at[pl.ds(row0, _RPT)], zbuf8)
      pltpu.sync_copy(zbuf8, deg_out.at[cid, pl.ds(row0, _RPT)])

  return body


_BLK = 5000


def _mm2_body(x_ref, wn_ref, xna_ref, xnb_ref):
  xn = jnp.dot(x_ref[...], wn_ref[...], preferred_element_type=jnp.float32)
  xna_ref[...] = xn[:, :_W]
  xnb_ref[...] = xn[:, _W:]


def _mid_body(x_ref, ws1_ref, b1_ref, acca_ref, accb_ref, deg_ref,
              ws2_ref, wn2_ref, b2_ref, hs2_ref, hn2_ref):
  agga = acca_ref[0] + acca_ref[1]
  aggb = accb_ref[0] + accb_ref[1]
  deg = deg_ref[0, :, 0:1] + deg_ref[1, :, 0:1]
  dinv = 1.0 / jnp.maximum(deg, 1.0)
  agg = jnp.concatenate([agga, aggb], axis=1)
  h = (jnp.dot(x_ref[...], ws1_ref[...], preferred_element_type=jnp.float32)
       + b1_ref[...] + agg * dinv)
  h = _SELU_SCALE * jnp.where(
      h > 0, h, _SELU_ALPHA * (jnp.exp(jnp.minimum(h, 0.0)) - 1.0))
  hs2_ref[...] = (
      jnp.dot(h, ws2_ref[...], preferred_element_type=jnp.float32)
      + b2_ref[...])
  hn2_ref[...] = jnp.dot(h, wn2_ref[...], preferred_element_type=jnp.float32)


def _out_body(hs2_ref, acc2_ref, deg_ref, o_ref):
  deg = deg_ref[0, :, 0:1] + deg_ref[1, :, 0:1]
  dinv = 1.0 / jnp.maximum(deg, 1.0)
  z = hs2_ref[...] + (acc2_ref[0] + acc2_ref[1]) * dinv
  m = jnp.max(z, axis=1, keepdims=True)
  e = jnp.exp(z - m)
  o_ref[...] = e / jnp.sum(e, axis=1, keepdims=True)


def _row_spec(w):
  return pl.BlockSpec((_BLK, w), lambda i: (i, 0))


def _part_spec(w):
  return pl.BlockSpec((2, _BLK, w), lambda i: (0, i, 0))


def _full_spec(r, c):
  return pl.BlockSpec((r, c), lambda i: (0, 0))


def kernel(x, edge_index, W_self1, W_neigh1, b1, W_self2, W_neigh2, b2):
  npad = _EP - _E
  pad_src = (jnp.arange(npad, dtype=jnp.int32) * 1009) % _N
  pad_dst = _N + (jnp.arange(npad, dtype=jnp.int32) % _NPAD)
  eidx = jnp.concatenate(
      [edge_index, jnp.stack([pad_src, pad_dst])], axis=1)

  xn1a, xn1b = pl.pallas_call(
      _mm2_body,
      grid=(_N // _BLK,),
      in_specs=[_row_spec(_D), _full_spec(_D, _H)],
      out_specs=[_row_spec(_W), _row_spec(_W)],
      out_shape=[jax.ShapeDtypeStruct((_N, _W), jnp.float32),
                 jax.ShapeDtypeStruct((_N, _W), jnp.float32)],
  )(x, W_neigh1)

  aux = jnp.concatenate([jnp.zeros((_RPT, 8), jnp.float32),
                         jnp.ones((_CHUNK, 8), jnp.float32)])
  agg_deg = _make_sc_agg(True, 2)
  agg_plain = _make_sc_agg(False, 4)
  accpa, degp = agg_deg(xn1a, eidx, aux)
  (accpb,) = agg_plain(xn1b, eidx)

  hs2, hn2 = pl.pallas_call(
      _mid_body,
      grid=(_N // _BLK,),
      in_specs=[_row_spec(_D), _full_spec(_D, _H), _full_spec(1, _H),
                _part_spec(_W), _part_spec(_W), _part_spec(8),
                _full_spec(_H, _C), _full_spec(_H, _C), _full_spec(1, _C)],
      out_specs=[_row_spec(_C), _row_spec(_C)],
      out_shape=[jax.ShapeDtypeStruct((_N, _C), jnp.float32),
                 jax.ShapeDtypeStruct((_N, _C), jnp.float32)],
  )(x, W_self1, b1.reshape(1, _H), accpa, accpb, degp,
    W_self2, W_neigh2, b2.reshape(1, _C))

  (accp2,) = agg_plain(hn2, eidx)

  out = pl.pallas_call(
      _out_body,
      grid=(_N // _BLK,),
      in_specs=[_row_spec(_C), _part_spec(_C), _part_spec(8)],
      out_specs=_row_spec(_C),
      out_shape=jax.ShapeDtypeStruct((_N, _C), jnp.float32),
  )(hs2, accp2, degp)
  return out
